# baseline (speedup 1.0000x reference)
import functools

import jax
import jax.numpy as jnp
from jax import lax
from jax.experimental import pallas as pl
from jax.experimental.pallas import tpu as pltpu
from jax.experimental.pallas import tpu_sc as plsc

B = 64
S = 1024
D = 768
N = B * S
L = 16
KD = D // L

NUM_WORKERS = 32
PPW = S // NUM_WORKERS
ROWJ = 8


def _add_pos(rows_v, pos_v, buf, t):
    def body(j0, carry):
        for j in range(ROWJ):
            for k in range(KD):
                sl = pl.ds(k * L, L)
                plsc.addupdate(rows_v.at[buf, j0 * ROWJ + j, sl],
                               pos_v[t, sl])
        return carry

    lax.fori_loop(0, B // ROWJ, body, 0)


def _store_out_idx(out_idx_v, buf, wbase, t):
    for m in range(B // L):
        vec = (lax.iota(jnp.int32, L) + (m * L)) * S + (wbase + t)
        out_idx_v[buf, pl.ds(m * L, L)] = vec


def _emb_body(xt_hbm, tok_hbm, pos_hbm, out_hbm,
              idx_v, rows_v, pos_v, out_idx_v, g0, g1, o0, o1):
    wid = lax.axis_index("s") * 2 + lax.axis_index("c")
    wbase = wid * PPW
    gsem = (g0, g1)
    osem = (o0, o1)

    pltpu.sync_copy(xt_hbm.at[pl.ds(wbase * B, PPW * B)], idx_v)
    pltpu.sync_copy(pos_hbm.at[pl.ds(wbase, PPW)], pos_v)

    def issue_gather(t, buf):
        pltpu.async_copy(tok_hbm.at[idx_v.at[pl.ds(t * B, B)]],
                         rows_v.at[buf], gsem[buf])

    def wait_gather(buf):
        pltpu.make_async_copy(tok_hbm.at[idx_v.at[pl.ds(0, B)]],
                              rows_v.at[buf], gsem[buf]).wait()

    def issue_store(buf):
        pltpu.async_copy(rows_v.at[buf], out_hbm.at[out_idx_v.at[buf]],
                         osem[buf])

    def wait_store(buf):
        pltpu.make_async_copy(rows_v.at[buf], out_hbm.at[out_idx_v.at[buf]],
                              osem[buf]).wait()

    issue_gather(0, 0)
    issue_gather(1, 1)
    wait_gather(0)
    _add_pos(rows_v, pos_v, 0, 0)
    _store_out_idx(out_idx_v, 0, wbase, 0)
    issue_store(0)

    def pair(tt, carry):
        t = 2 * tt + 1
        wait_gather(1)
        _add_pos(rows_v, pos_v, 1, t)
        wait_store(0)
        issue_gather(t + 1, 0)
        _store_out_idx(out_idx_v, 1, wbase, t)
        issue_store(1)

        wait_gather(0)
        _add_pos(rows_v, pos_v, 0, t + 1)
        wait_store(1)
        issue_gather(t + 2, 1)
        _store_out_idx(out_idx_v, 0, wbase, t + 1)
        issue_store(0)
        return carry

    lax.fori_loop(0, (PPW - 2) // 2, pair, 0)

    wait_gather(1)
    _add_pos(rows_v, pos_v, 1, PPW - 1)
    wait_store(0)
    _store_out_idx(out_idx_v, 1, wbase, PPW - 1)
    issue_store(1)
    wait_store(1)


@jax.jit
def _emb(x_t, token_emb, pos2d):
    mesh = plsc.VectorSubcoreMesh(core_axis_name="c", subcore_axis_name="s")
    f = functools.partial(
        pl.kernel,
        out_type=jax.ShapeDtypeStruct((N, D), jnp.float32),
        mesh=mesh,
        scratch_types=[
            pltpu.VMEM((PPW * B,), jnp.int32),
            pltpu.VMEM((2, B, D), jnp.float32),
            pltpu.VMEM((PPW, D), jnp.float32),
            pltpu.VMEM((2, B), jnp.int32),
            pltpu.SemaphoreType.DMA,
            pltpu.SemaphoreType.DMA,
            pltpu.SemaphoreType.DMA,
            pltpu.SemaphoreType.DMA,
        ],
    )(_emb_body)
    return f(x_t, token_emb, pos2d)


def kernel(x, token_emb, pos_emb):
    x_t = x.T.reshape(N)
    pos2d = pos_emb.reshape(S, D)
    out = _emb(x_t, token_emb, pos2d)
    return out.reshape(B, S, D)


# speedup vs baseline: 1.1166x; 1.1166x over previous
import functools

import jax
import jax.numpy as jnp
from jax import lax
from jax.experimental import pallas as pl
from jax.experimental.pallas import tpu as pltpu
from jax.experimental.pallas import tpu_sc as plsc

B = 64
S = 1024
D = 768
N = B * S
L = 16
KD = D // L

NUM_WORKERS = 32
PPW = S // NUM_WORKERS
ROWJ = 8


def _add_pos(rows_v, pos_v, buf, t):
    rows2d = rows_v.at[buf]

    def body(j, carry):
        for k in range(KD):
            sl = pl.ds(k * L, L)
            plsc.addupdate(rows2d.at[j, sl], pos_v[t, sl])
        return carry

    lax.fori_loop(0, B, body, 0)


def _store_out_idx(out_idx_v, buf, wbase, t):
    for m in range(B // L):
        vec = (lax.iota(jnp.int32, L) + (m * L)) * S + (wbase + t)
        out_idx_v[buf, pl.ds(m * L, L)] = vec


def _emb_body(xt_hbm, tok_hbm, pos_hbm, out_hbm,
              idx_v, rows_v, pos_v, out_idx_v, g0, g1, o0, o1):
    wid = lax.axis_index("s") * 2 + lax.axis_index("c")
    wbase = wid * PPW
    gsem = (g0, g1)
    osem = (o0, o1)

    pltpu.sync_copy(xt_hbm.at[pl.ds(wbase * B, PPW * B)], idx_v)
    pltpu.sync_copy(pos_hbm.at[pl.ds(wbase, PPW)], pos_v)

    def issue_gather(t, buf):
        pltpu.async_copy(tok_hbm.at[idx_v.at[pl.ds(t * B, B)]],
                         rows_v.at[buf], gsem[buf])

    def wait_gather(buf):
        pltpu.make_async_copy(tok_hbm.at[idx_v.at[pl.ds(0, B)]],
                              rows_v.at[buf], gsem[buf]).wait()

    def issue_store(buf):
        pltpu.async_copy(rows_v.at[buf], out_hbm.at[out_idx_v.at[buf]],
                         osem[buf])

    def wait_store(buf):
        pltpu.make_async_copy(rows_v.at[buf], out_hbm.at[out_idx_v.at[buf]],
                              osem[buf]).wait()

    issue_gather(0, 0)
    issue_gather(1, 1)
    wait_gather(0)
    _add_pos(rows_v, pos_v, 0, 0)
    _store_out_idx(out_idx_v, 0, wbase, 0)
    issue_store(0)

    def pair(tt, carry):
        t = 2 * tt + 1
        wait_gather(1)
        wait_store(0)
        issue_gather(t + 1, 0)
        _add_pos(rows_v, pos_v, 1, t)
        _store_out_idx(out_idx_v, 1, wbase, t)
        issue_store(1)

        wait_gather(0)
        wait_store(1)
        issue_gather(t + 2, 1)
        _add_pos(rows_v, pos_v, 0, t + 1)
        _store_out_idx(out_idx_v, 0, wbase, t + 1)
        issue_store(0)
        return carry

    lax.fori_loop(0, (PPW - 2) // 2, pair, 0)

    wait_gather(1)
    _add_pos(rows_v, pos_v, 1, PPW - 1)
    wait_store(0)
    _store_out_idx(out_idx_v, 1, wbase, PPW - 1)
    issue_store(1)
    wait_store(1)


@jax.jit
def _emb(x_t, token_emb, pos2d):
    mesh = plsc.VectorSubcoreMesh(core_axis_name="c", subcore_axis_name="s")
    f = functools.partial(
        pl.kernel,
        out_type=jax.ShapeDtypeStruct((N, D), jnp.float32),
        mesh=mesh,
        scratch_types=[
            pltpu.VMEM((PPW * B,), jnp.int32),
            pltpu.VMEM((2, B, D), jnp.float32),
            pltpu.VMEM((PPW, D), jnp.float32),
            pltpu.VMEM((2, B), jnp.int32),
            pltpu.SemaphoreType.DMA,
            pltpu.SemaphoreType.DMA,
            pltpu.SemaphoreType.DMA,
            pltpu.SemaphoreType.DMA,
        ],
    )(_emb_body)
    return f(x_t, token_emb, pos2d)


def kernel(x, token_emb, pos_emb):
    x_t = x.T.reshape(N)
    pos2d = pos_emb.reshape(S, D)
    out = _emb(x_t, token_emb, pos2d)
    return out.reshape(B, S, D)
